# baseline (speedup 1.0000x reference)
import functools

import jax
import jax.numpy as jnp
from jax import lax
from jax.experimental import pallas as pl
from jax.experimental.pallas import tpu as pltpu
from jax.experimental.pallas import tpu_sc as plsc

BATCH = 16384
D = 64

_info = plsc.get_sparse_core_info()
NC, NS, L = _info.num_cores, _info.num_subcores, _info.num_lanes
NW = NC * NS
BPW = BATCH // NW
CHUNK = 128
NCHUNK = BPW // CHUNK
GPC = CHUNK // 16


def _mf_body(uid_hbm, iid_hbm, ut_hbm, it_hbm, out_hbm,
             uidx_v, iidx_v, uh_v, ih_v, ubuf, ibuf, out_v, sem):
    wid = lax.axis_index("s") * NC + lax.axis_index("c")
    base = wid * BPW

    for c in range(NCHUNK):
        pltpu.sync_copy(uid_hbm.at[pl.ds(base + c * CHUNK, CHUNK)], uidx_v.at[c])
        pltpu.sync_copy(iid_hbm.at[pl.ds(base + c * CHUNK, CHUNK)], iidx_v.at[c])

    for c in range(NCHUNK):
        for j in range(CHUNK // L):
            s = pl.ds(j * L, L)
            uh_v[c, s] = jax.lax.shift_right_logical(uidx_v[c, s], 1)
            ih_v[c, s] = jax.lax.shift_right_logical(iidx_v[c, s], 1)

    def fire(c):
        slot = c % 2
        return (
            pltpu.async_copy(ut_hbm.at[uh_v.at[c]], ubuf.at[slot], sem),
            pltpu.async_copy(it_hbm.at[ih_v.at[c]], ibuf.at[slot], sem),
        )

    lanes = lax.broadcasted_iota(jnp.int32, (L,), 0)

    def compute(c):
        slot = c % 2
        for g in range(GPC):
            s = pl.ds(g * L, L)
            ucol = (uidx_v[c, s] & 1) * D
            icol = (iidx_v[c, s] & 1) * D
            rows = g * L + lanes
            acc = jnp.zeros((L,), jnp.float32)
            for d in range(D):
                u = plsc.load_gather(ubuf.at[slot], [rows, ucol + d])
                v = plsc.load_gather(ibuf.at[slot], [rows, icol + d])
                acc = acc + u * v
            out_v[pl.ds(c * CHUNK + g * L, L)] = acc

    pending = fire(0)
    for c in range(NCHUNK):
        nxt = fire(c + 1) if c + 1 < NCHUNK else ()
        for cp in pending:
            cp.wait()
        compute(c)
        pending = nxt

    pltpu.sync_copy(out_v, out_hbm.at[pl.ds(base, BPW)])


@functools.partial(
    pl.kernel,
    out_type=jax.ShapeDtypeStruct((BATCH,), jnp.float32),
    mesh=plsc.VectorSubcoreMesh(core_axis_name="c", subcore_axis_name="s"),
    compiler_params=pltpu.CompilerParams(needs_layout_passes=False),
    scratch_types=[
        pltpu.VMEM((NCHUNK, CHUNK), jnp.int32),
        pltpu.VMEM((NCHUNK, CHUNK), jnp.int32),
        pltpu.VMEM((NCHUNK, CHUNK), jnp.int32),
        pltpu.VMEM((NCHUNK, CHUNK), jnp.int32),
        pltpu.VMEM((2, CHUNK, 2 * D), jnp.float32),
        pltpu.VMEM((2, CHUNK, 2 * D), jnp.float32),
        pltpu.VMEM((BPW,), jnp.float32),
        pltpu.SemaphoreType.DMA,
    ],
)
def _mf_kernel(uid_hbm, iid_hbm, ut_hbm, it_hbm, out_hbm,
               uidx_v, iidx_v, uh_v, ih_v, ubuf, ibuf, out_v, sem):
    _mf_body(uid_hbm, iid_hbm, ut_hbm, it_hbm, out_hbm,
             uidx_v, iidx_v, uh_v, ih_v, ubuf, ibuf, out_v, sem)


def kernel(user_ids, item_ids, user_table, item_table):
    ut2 = user_table.reshape(user_table.shape[0] // 2, 2 * D)
    it2 = item_table.reshape(item_table.shape[0] // 2, 2 * D)
    return _mf_kernel(user_ids.astype(jnp.int32), item_ids.astype(jnp.int32),
                      ut2, it2)
